# Initial kernel scaffold; baseline (speedup 1.0000x reference)
import jax
import jax.numpy as jnp
from jax.experimental import pallas as pl


def kernel(num_x, num_mask, x, edge_index, W_num, b_num, a_in, W_node, b_node, W1, b1, a1, W2, b2, a2, W3, b3):
    raise NotImplementedError("write your pallas kernel here")


# speedup vs baseline: 12.5024x; 12.5024x over previous
import functools

import jax
import jax.numpy as jnp
from jax import lax
from jax.experimental import pallas as pl
from jax.experimental.pallas import tpu as pltpu
from jax.experimental.pallas import tpu_sc as plsc

N = 10000
EMBED = 128
HIDDEN = 128
NCLS = 40
NCLS_PAD = 128

NC = 2
NS = 16
NW = NC * NS
CHUNK = 128

N_PAD = 10112
STRIPE = N_PAD // NS
TRASH = N


def _sc_prop_body(n_chunks, u_hbm, src_hbm, dst_hbm, zeros_hbm, out_hbm,
                  idx_s, idx_d, rows, acc, sem):
    c = lax.axis_index("c")
    s = lax.axis_index("s")
    base = s * STRIPE
    pltpu.sync_copy(zeros_hbm.at[pl.ds(base, STRIPE)],
                    acc.at[pl.ds(base, STRIPE)])
    pltpu.sync_copy(src_hbm.at[c, s], idx_s)
    pltpu.sync_copy(dst_hbm.at[c, s], idx_d)
    plsc.subcore_barrier()

    def body(j, carry):
        pltpu.async_copy(u_hbm.at[idx_s.at[j]], rows, sem).wait()
        pltpu.sync_copy(rows, acc.at[idx_d.at[j]], add=True)
        return carry

    lax.fori_loop(0, n_chunks, body, 0)
    plsc.subcore_barrier()
    pltpu.sync_copy(acc.at[pl.ds(base, STRIPE)],
                    out_hbm.at[c, pl.ds(base, STRIPE)])


def _sc_deg_body(n_chunks, dst_hbm, ones_hbm, zeros_hbm, out_hbm,
                 idx_d, ones_v, acc):
    c = lax.axis_index("c")
    s = lax.axis_index("s")
    base = s * STRIPE
    pltpu.sync_copy(zeros_hbm.at[pl.ds(base, STRIPE)],
                    acc.at[pl.ds(base, STRIPE)])
    pltpu.sync_copy(dst_hbm.at[c, s], idx_d)
    pltpu.sync_copy(ones_hbm, ones_v)
    plsc.subcore_barrier()

    def body(j, carry):
        pltpu.sync_copy(ones_v, acc.at[idx_d.at[j]], add=True)
        return carry

    lax.fori_loop(0, n_chunks, body, 0)
    plsc.subcore_barrier()
    pltpu.sync_copy(acc.at[pl.ds(base, STRIPE)],
                    out_hbm.at[c, pl.ds(base, STRIPE)])


def _sc_propagate(u, src3, dst3, zeros, n_chunks, d):
    mesh = plsc.VectorSubcoreMesh(core_axis_name="c", subcore_axis_name="s")
    return pl.kernel(
        functools.partial(_sc_prop_body, n_chunks),
        out_type=jax.ShapeDtypeStruct((NC, N_PAD, d), jnp.float32),
        mesh=mesh,
        scratch_types=[
            pltpu.VMEM((n_chunks, CHUNK), jnp.int32),
            pltpu.VMEM((n_chunks, CHUNK), jnp.int32),
            pltpu.VMEM((CHUNK, d), jnp.float32),
            pltpu.VMEM_SHARED((N_PAD, d), jnp.float32),
            pltpu.SemaphoreType.DMA,
        ],
    )(u, src3, dst3, zeros)


def _sc_degree(dst3, ones, zeros, n_chunks):
    mesh = plsc.VectorSubcoreMesh(core_axis_name="c", subcore_axis_name="s")
    return pl.kernel(
        functools.partial(_sc_deg_body, n_chunks),
        out_type=jax.ShapeDtypeStruct((NC, N_PAD, HIDDEN), jnp.float32),
        mesh=mesh,
        scratch_types=[
            pltpu.VMEM((n_chunks, CHUNK), jnp.int32),
            pltpu.VMEM((CHUNK, HIDDEN), jnp.float32),
            pltpu.VMEM_SHARED((N_PAD, HIDDEN), jnp.float32),
        ],
    )(dst3, ones, zeros)


BN = 400


def _prelu(x, a):
    return jnp.where(x > 0, x, a * x)


def _tc_in_body(deg_ref, num_x_ref, num_mask_ref, x_ref, w_num_ref, b_num_ref,
                a_in_ref, w_node_ref, b_node_ref, w1_ref, u1_ref, dinv_ref):
    deg = deg_ref[0, :, 0:1] + deg_ref[1, :, 0:1]
    dinv = jnp.where(deg > 0, lax.rsqrt(deg), 0.0)
    h = _prelu((num_x_ref[...] * num_mask_ref[...]) * w_num_ref[...]
               + b_num_ref[...], a_in_ref[...])
    h = h + jnp.dot(x_ref[...], w_node_ref[...],
                    preferred_element_type=jnp.float32) + b_node_ref[...]
    u1_ref[...] = jnp.dot(h, w1_ref[...],
                          preferred_element_type=jnp.float32) * dinv
    dinv_ref[...] = dinv


def _tc_mid_body(agg_ref, dinv_ref, b_ref, a_ref, w_ref, u_ref):
    dinv = dinv_ref[...]
    g = (agg_ref[0] + agg_ref[1]) * dinv + b_ref[...]
    h = _prelu(g, a_ref[...])
    u_ref[...] = jnp.dot(h, w_ref[...],
                         preferred_element_type=jnp.float32) * dinv


def _tc_out_body(agg_ref, dinv_ref, b_ref, out_ref):
    t = ((agg_ref[0, :, :NCLS] + agg_ref[1, :, :NCLS]) * dinv_ref[...]
         + b_ref[...])
    m = jnp.max(t, axis=1, keepdims=True)
    e = jnp.exp(t - m)
    lse = jnp.log(jnp.sum(e, axis=1, keepdims=True))
    out_ref[...] = t - m - lse


def _full(shape):
    return pl.BlockSpec(shape, lambda i: (0,) * len(shape))


def _rows(shape):
    return pl.BlockSpec(shape, lambda i: (i,) + (0,) * (len(shape) - 1))


def _tc_input_proj(deg2, num_x, num_mask, x, w_num, b_num, a_in, w_node,
                   b_node, w1):
    return pl.pallas_call(
        _tc_in_body,
        grid=(N // BN,),
        in_specs=[
            pl.BlockSpec((NC, BN, HIDDEN), lambda i: (0, i, 0)),
            _rows((BN, 1)), _rows((BN, 1)), _rows((BN, EMBED)),
            _full((1, EMBED)), _full((1, EMBED)), _full((1, EMBED)),
            _full((EMBED, HIDDEN)), _full((1, HIDDEN)),
            _full((HIDDEN, HIDDEN)),
        ],
        out_specs=[_rows((BN, HIDDEN)), _rows((BN, 1))],
        out_shape=[jax.ShapeDtypeStruct((N, HIDDEN), jnp.float32),
                   jax.ShapeDtypeStruct((N, 1), jnp.float32)],
    )(deg2, num_x, num_mask, x, w_num, b_num, a_in, w_node, b_node, w1)


def _tc_mid(agg, dinv, b, a, w):
    dout = w.shape[1]
    return pl.pallas_call(
        _tc_mid_body,
        grid=(N // BN,),
        in_specs=[
            pl.BlockSpec((NC, BN, HIDDEN), lambda i: (0, i, 0)),
            _rows((BN, 1)), _full((1, HIDDEN)), _full((1, HIDDEN)),
            _full((HIDDEN, dout)),
        ],
        out_specs=_rows((BN, dout)),
        out_shape=jax.ShapeDtypeStruct((N, dout), jnp.float32),
    )(agg, dinv, b, a, w)


def _tc_logits(agg, dinv, b3):
    return pl.pallas_call(
        _tc_out_body,
        grid=(N // BN,),
        in_specs=[
            pl.BlockSpec((NC, BN, NCLS_PAD), lambda i: (0, i, 0)),
            _rows((BN, 1)), _full((1, NCLS)),
        ],
        out_specs=_rows((BN, NCLS)),
        out_shape=jax.ShapeDtypeStruct((N, NCLS), jnp.float32),
    )(agg, dinv, b3)


def kernel(num_x, num_mask, x, edge_index, W_num, b_num, a_in, W_node,
           b_node, W1, b1, a1, W2, b2, a2, W3, b3):
    e = edge_index.shape[1]
    et = e + N
    n_chunks = -(-et // (NW * CHUNK))
    et_pad = NW * CHUNK * n_chunks

    loop = jnp.arange(N, dtype=edge_index.dtype)
    src = jnp.concatenate([edge_index[0], loop,
                           jnp.zeros((et_pad - et,), edge_index.dtype)])
    dst = jnp.concatenate([edge_index[1], loop,
                           jnp.full((et_pad - et,), TRASH, edge_index.dtype)])
    src3 = src.reshape(NC, NS, n_chunks, CHUNK)
    dst3 = dst.reshape(NC, NS, n_chunks, CHUNK)

    zeros128 = jnp.zeros((N_PAD, HIDDEN), jnp.float32)
    ones128 = jnp.ones((CHUNK, HIDDEN), jnp.float32)

    deg2 = _sc_degree(dst3, ones128, zeros128, n_chunks)

    u1, dinv = _tc_input_proj(
        deg2, num_x, num_mask, x, W_num, b_num.reshape(1, -1),
        a_in.reshape(1, -1), W_node, b_node.reshape(1, -1), W1)

    agg1 = _sc_propagate(u1, src3, dst3, zeros128, n_chunks, HIDDEN)
    u2 = _tc_mid(agg1, dinv, b1.reshape(1, -1), a1.reshape(1, -1), W2)

    agg2 = _sc_propagate(u2, src3, dst3, zeros128, n_chunks, HIDDEN)
    w3p = jnp.zeros((HIDDEN, NCLS_PAD), jnp.float32).at[:, :NCLS].set(W3)
    u3 = _tc_mid(agg2, dinv, b2.reshape(1, -1), a2.reshape(1, -1), w3p)

    agg3 = _sc_propagate(u3, src3, dst3, zeros128, n_chunks, NCLS_PAD)
    return _tc_logits(agg3, dinv, b3.reshape(1, -1))
